# baseline (speedup 1.0000x reference)
import jax
import jax.numpy as jnp
from jax.experimental import pallas as pl

BATCH = 4096
MAXLEN = 200
EMBED_DIM = 64
ROW = MAXLEN * EMBED_DIM

BB = 256


def _add_kernel(x_ref, pos_ref, o_ref):
    o_ref[...] = x_ref[...] + pos_ref[...]


def kernel(x, pos_table):
    x2 = x.reshape(BATCH, ROW)
    pos2 = pos_table.reshape(1, ROW)
    out = pl.pallas_call(
        _add_kernel,
        grid=(BATCH // BB,),
        in_specs=[
            pl.BlockSpec((BB, ROW), lambda i: (i, 0)),
            pl.BlockSpec((1, ROW), lambda i: (0, 0)),
        ],
        out_specs=pl.BlockSpec((BB, ROW), lambda i: (i, 0)),
        out_shape=jax.ShapeDtypeStruct((BATCH, ROW), jnp.float32),
    )(x2, pos2)
    return out.reshape(BATCH, MAXLEN, EMBED_DIM)
